# baseline (speedup 1.0000x reference)
import jax
import jax.numpy as jnp
from jax.experimental import pallas as pl

KNN_K = 20
BR = 256
BIG = 3.0e38


def _knn_body(xr_ref, xa_ref, out_ref):
    xr = xr_ref[0]
    xa = xa_ref[0]
    n = xa.shape[0]
    inner = jax.lax.dot_general(
        xr, xa, (((1,), (1,)), ((), ())),
        preferred_element_type=jnp.float32)
    xxr = jnp.sum(xr * xr, axis=1, keepdims=True)
    xxa = jnp.sum(xa * xa, axis=1)
    d = (xxr - 2.0 * inner) + xxa[None, :]

    iota = jax.lax.broadcasted_iota(jnp.int32, d.shape, 1).astype(jnp.float32)
    cols = []
    for _ in range(KNN_K):
        m = jnp.min(d, axis=1, keepdims=True)
        cand = jnp.where(d == m, iota, jnp.float32(n))
        j = jnp.min(cand, axis=1, keepdims=True)
        cols.append(j)
        d = jnp.where(cand == j, BIG, d)
    out_ref[0] = jnp.concatenate(cols, axis=1).astype(jnp.int32)


def _knn_call(x):
    b, n, dd = x.shape
    return pl.pallas_call(
        _knn_body,
        grid=(b, n // BR),
        in_specs=[
            pl.BlockSpec((1, BR, dd), lambda bi, ri: (bi, ri, 0)),
            pl.BlockSpec((1, n, dd), lambda bi, ri: (bi, 0, 0)),
        ],
        out_specs=pl.BlockSpec((1, BR, KNN_K), lambda bi, ri: (bi, ri, 0)),
        out_shape=jax.ShapeDtypeStruct((b, n, KNN_K), jnp.int32),
    )(x, x)


def kernel(x):
    b = x.shape[0]
    devs = jax.devices()
    nd = 2 if (len(devs) >= 2 and b % 2 == 0) else 1
    if nd == 1:
        return (x, _knn_call(x))
    import numpy as np
    from jax.sharding import Mesh, PartitionSpec as P
    try:
        from jax import shard_map as _shard_map
    except ImportError:
        from jax.experimental.shard_map import shard_map as _shard_map
    mesh = Mesh(np.asarray(devs[:nd]), ("d",))
    idx = _shard_map(
        _knn_call, mesh=mesh, in_specs=(P("d"),), out_specs=P("d"),
        check_vma=False,
    )(x)
    return (x, idx)


# speedup vs baseline: 1.7666x; 1.7666x over previous
import jax
import jax.numpy as jnp
from jax.experimental import pallas as pl

KNN_K = 20
BR = 256
DEPTH = 5
BIG = 3.0e38


def _topk_argmin(d, n):
    iota = jax.lax.broadcasted_iota(jnp.int32, d.shape, 1)
    cols = []
    for _ in range(KNN_K):
        j = jnp.argmin(d, axis=1, keepdims=True)
        cols.append(j)
        d = jnp.where(iota == j, BIG, d)
    return jnp.concatenate(cols, axis=1)


def _knn_body(xr_ref, xa_ref, out_ref):
    xr = xr_ref[0]
    xa = xa_ref[0]
    n = xa.shape[0]
    ntiles = n // 128
    inner = jax.lax.dot_general(
        xr, xa, (((1,), (1,)), ((), ())),
        preferred_element_type=jnp.float32)
    xxr = jnp.sum(xr * xr, axis=1, keepdims=True)
    xxa = jnp.sum(xa * xa, axis=1)
    d = (xxr - 2.0 * inner) + xxa[None, :]

    lane = jax.lax.broadcasted_iota(jnp.int32, (BR, 128), 1).astype(jnp.float32)
    vals = [jnp.full((BR, 128), BIG, jnp.float32) for _ in range(DEPTH)]
    args = [jnp.zeros((BR, 128), jnp.float32) for _ in range(DEPTH)]
    for g in range(ntiles):
        v = d[:, g * 128:(g + 1) * 128]
        a = lane + float(g * 128)
        for l in range(DEPTH):
            c = v < vals[l]
            vals[l], v = jnp.where(c, v, vals[l]), jnp.where(c, vals[l], v)
            args[l], a = jnp.where(c, a, args[l]), jnp.where(c, args[l], a)

    tail_v, tail_g = vals[DEPTH - 1], args[DEPTH - 1]

    cols = []
    for _ in range(KNN_K):
        m = jnp.min(vals[0], axis=1, keepdims=True)
        cand = jnp.where(vals[0] == m, args[0], jnp.float32(n))
        j = jnp.min(cand, axis=1, keepdims=True)
        cols.append(j)
        msk = cand == j
        for l in range(DEPTH - 1):
            vals[l] = jnp.where(msk, vals[l + 1], vals[l])
            args[l] = jnp.where(msk, args[l + 1], args[l])
        vals[DEPTH - 1] = jnp.where(msk, BIG, vals[DEPTH - 1])
    fast = jnp.concatenate(cols, axis=1)
    out_ref[0] = fast.astype(jnp.int32)

    v20, i20 = m, cols[KNN_K - 1]
    hidden = (tail_v < v20) | ((tail_v == v20) & (tail_g < i20))
    bad = jnp.any(hidden)

    @pl.when(bad)
    def _fallback():
        out_ref[0] = _topk_argmin(d, n).astype(jnp.int32)


def _knn_call(x):
    b, n, dd = x.shape
    return pl.pallas_call(
        _knn_body,
        grid=(b, n // BR),
        in_specs=[
            pl.BlockSpec((1, BR, dd), lambda bi, ri: (bi, ri, 0)),
            pl.BlockSpec((1, n, dd), lambda bi, ri: (bi, 0, 0)),
        ],
        out_specs=pl.BlockSpec((1, BR, KNN_K), lambda bi, ri: (bi, ri, 0)),
        out_shape=jax.ShapeDtypeStruct((b, n, KNN_K), jnp.int32),
    )(x, x)


def kernel(x):
    return (x, _knn_call(x))


# speedup vs baseline: 1.9352x; 1.0954x over previous
import jax
import jax.numpy as jnp
from jax.experimental import pallas as pl

KNN_K = 20
BR = 512
DEPTH = 5
BIG = 3.0e38


def _topk_argmin(d, n):
    iota = jax.lax.broadcasted_iota(jnp.int32, d.shape, 1)
    cols = []
    for _ in range(KNN_K):
        j = jnp.argmin(d, axis=1, keepdims=True)
        cols.append(j)
        d = jnp.where(iota == j, BIG, d)
    return jnp.concatenate(cols, axis=1)


def _knn_body(xr_ref, xa_ref, out_ref):
    xr = xr_ref[0]
    xa = xa_ref[0]
    n = xa.shape[0]
    ntiles = n // 128
    inner = jax.lax.dot_general(
        xr, xa, (((1,), (1,)), ((), ())),
        preferred_element_type=jnp.float32)
    xxr = jnp.sum(xr * xr, axis=1, keepdims=True)
    xxa = jnp.sum(xa * xa, axis=1)
    d = (xxr - 2.0 * inner) + xxa[None, :]

    lane = jax.lax.broadcasted_iota(jnp.int32, (BR, 128), 1).astype(jnp.float32)
    vals = [jnp.full((BR, 128), BIG, jnp.float32) for _ in range(DEPTH)]
    args = [jnp.zeros((BR, 128), jnp.float32) for _ in range(DEPTH)]
    for g in range(ntiles):
        v = d[:, g * 128:(g + 1) * 128]
        a = lane + float(g * 128)
        for l in range(DEPTH):
            c = v < vals[l]
            vals[l], v = jnp.where(c, v, vals[l]), jnp.where(c, vals[l], v)
            args[l], a = jnp.where(c, a, args[l]), jnp.where(c, args[l], a)

    tail_v, tail_g = vals[DEPTH - 1], args[DEPTH - 1]

    cols = []
    for _ in range(KNN_K):
        m = jnp.min(vals[0], axis=1, keepdims=True)
        cand = jnp.where(vals[0] == m, args[0], jnp.float32(n))
        j = jnp.min(cand, axis=1, keepdims=True)
        cols.append(j)
        msk = cand == j
        for l in range(DEPTH - 1):
            vals[l] = jnp.where(msk, vals[l + 1], vals[l])
            args[l] = jnp.where(msk, args[l + 1], args[l])
        vals[DEPTH - 1] = jnp.where(msk, BIG, vals[DEPTH - 1])
    fast = jnp.concatenate(cols, axis=1)
    out_ref[0] = fast.astype(jnp.int32)

    v20, i20 = m, cols[KNN_K - 1]
    hidden = (tail_v < v20) | ((tail_v == v20) & (tail_g < i20))
    bad = jnp.any(hidden)

    @pl.when(bad)
    def _fallback():
        out_ref[0] = _topk_argmin(d, n).astype(jnp.int32)


def _knn_call(x):
    b, n, dd = x.shape
    return pl.pallas_call(
        _knn_body,
        grid=(b, n // BR),
        in_specs=[
            pl.BlockSpec((1, BR, dd), lambda bi, ri: (bi, ri, 0)),
            pl.BlockSpec((1, n, dd), lambda bi, ri: (bi, 0, 0)),
        ],
        out_specs=pl.BlockSpec((1, BR, KNN_K), lambda bi, ri: (bi, ri, 0)),
        out_shape=jax.ShapeDtypeStruct((b, n, KNN_K), jnp.int32),
    )(x, x)


def kernel(x):
    return (x, _knn_call(x))


# speedup vs baseline: 1.9369x; 1.0009x over previous
import jax
import jax.numpy as jnp
from jax.experimental import pallas as pl

KNN_K = 20
BR = 512
DEPTH = 5
BIG = 3.0e38


def _topk_argmin(d, n):
    iota = jax.lax.broadcasted_iota(jnp.int32, d.shape, 1)
    cols = []
    for _ in range(KNN_K):
        j = jnp.argmin(d, axis=1, keepdims=True)
        cols.append(j)
        d = jnp.where(iota == j, BIG, d)
    return jnp.concatenate(cols, axis=1)


def _knn_body(xr_ref, xa_ref, out_ref):
    xr = xr_ref[0]
    xa = xa_ref[0]
    n = xa.shape[0]
    ntiles = n // 128
    inner = jax.lax.dot_general(
        xr, xa, (((1,), (1,)), ((), ())),
        preferred_element_type=jnp.float32)
    xxr = jnp.sum(xr * xr, axis=1, keepdims=True)
    xxa = jnp.sum(xa * xa, axis=1)

    lane = jax.lax.broadcasted_iota(jnp.int32, (BR, 128), 1).astype(jnp.float32)
    vals = [jnp.full((BR, 128), BIG, jnp.float32) for _ in range(DEPTH)]
    args = [jnp.zeros((BR, 128), jnp.float32) for _ in range(DEPTH)]
    for g in range(ntiles):
        v = (xxr - 2.0 * inner[:, g * 128:(g + 1) * 128]) + xxa[None, g * 128:(g + 1) * 128]
        a = lane + float(g * 128)
        for l in range(DEPTH):
            c = v < vals[l]
            vals[l], v = jnp.where(c, v, vals[l]), jnp.where(c, vals[l], v)
            args[l], a = jnp.where(c, a, args[l]), jnp.where(c, args[l], a)

    tail_v, tail_g = vals[DEPTH - 1], args[DEPTH - 1]

    cols = []
    for _ in range(KNN_K):
        m = jnp.min(vals[0], axis=1, keepdims=True)
        cand = jnp.where(vals[0] == m, args[0], jnp.float32(n))
        j = jnp.min(cand, axis=1, keepdims=True)
        cols.append(j)
        msk = cand == j
        for l in range(DEPTH - 1):
            vals[l] = jnp.where(msk, vals[l + 1], vals[l])
            args[l] = jnp.where(msk, args[l + 1], args[l])
        vals[DEPTH - 1] = jnp.where(msk, BIG, vals[DEPTH - 1])
    fast = jnp.concatenate(cols, axis=1)
    out_ref[0] = fast.astype(jnp.int32)

    v20, i20 = m, cols[KNN_K - 1]
    hidden = (tail_v < v20) | ((tail_v == v20) & (tail_g < i20))
    bad = jnp.any(hidden)

    @pl.when(bad)
    def _fallback():
        d = (xxr - 2.0 * inner) + xxa[None, :]
        out_ref[0] = _topk_argmin(d, n).astype(jnp.int32)


def _knn_call(x):
    b, n, dd = x.shape
    return pl.pallas_call(
        _knn_body,
        grid=(b, n // BR),
        in_specs=[
            pl.BlockSpec((1, BR, dd), lambda bi, ri: (bi, ri, 0)),
            pl.BlockSpec((1, n, dd), lambda bi, ri: (bi, 0, 0)),
        ],
        out_specs=pl.BlockSpec((1, BR, KNN_K), lambda bi, ri: (bi, ri, 0)),
        out_shape=jax.ShapeDtypeStruct((b, n, KNN_K), jnp.int32),
    )(x, x)


def kernel(x):
    return (x, _knn_call(x))
